# Initial kernel scaffold; baseline (speedup 1.0000x reference)
import jax
import jax.numpy as jnp
from jax.experimental import pallas as pl


def kernel(atoms, pos, edge_index, batch, emb, mlp_w1, mlp_b1, mlp_w2, mlp_b2, pred_w1, pred_b1, pred_w2, pred_b2):
    raise NotImplementedError("write your pallas kernel here")


# speedup vs baseline: 3.4187x; 3.4187x over previous
import functools
import math

import jax
import jax.numpy as jnp
from jax.experimental import pallas as pl
from jax.experimental.pallas import tpu as pltpu

_R_MAX = 10.0
_NUM_BESSEL = 8
_C = 32
_K = 9
_CK = _C * _K
_NUM_LAYERS = 3
_N = 10000
_E = 160000
_EB = 1000
_NB = _E // _EB
_G = 64
_RB = 1000
_NRB = _N // _RB


def _tfn_kernel(idx_ref, vec_ref, aoh_ref, boh_ref, emb_ref,
                w1_ref, b1_ref, w2_ref, b2_ref,
                pw1_ref, pb1_ref, pw2_ref, pb2_ref,
                out_ref, h_ref, agg_ref, hsrc_ref, msg_ref):
    l = pl.program_id(0)
    b = pl.program_id(1)

    def apply_update():
        for r in range(_NRB):
            rows = slice(r * _RB, (r + 1) * _RB)
            hold = h_ref[rows, :]
            ag = agg_ref[rows, :]
            s = ag[:, :_C]
            sg = jax.nn.sigmoid(s)
            parts = [s * sg + hold[:, :_C]]
            for k in range(1, _K):
                sl = slice(k * _C, (k + 1) * _C)
                parts.append(ag[:, sl] * sg + hold[:, sl])
            h_ref[rows, :] = jnp.concatenate(parts, axis=1)

    @pl.when(b == 0)
    def _():
        @pl.when(l == 0)
        def _():
            for r in range(_NRB):
                rows = slice(r * _RB, (r + 1) * _RB)
                h0 = jnp.dot(aoh_ref[rows, :], emb_ref[:],
                             preferred_element_type=jnp.float32)
                h_ref[rows, :] = jnp.concatenate(
                    [h0, jnp.zeros((_RB, _CK - _C), jnp.float32)], axis=1)

        @pl.when(l > 0)
        def _():
            apply_update()

        for r in range(_NRB):
            rows = slice(r * _RB, (r + 1) * _RB)
            agg_ref[rows, :] = jnp.zeros((_RB, _CK), jnp.float32)

    base = b * _EB

    def gather_body(i, carry):
        s = jax.lax.shift_right_logical(idx_ref[base + i], 14)
        hsrc_ref[pl.ds(i, 1), :] = h_ref[pl.ds(s, 1), :]
        return carry

    jax.lax.fori_loop(0, _EB, gather_body, 0)

    vec = vec_ref[:]
    x = vec[:, 0:1]
    y = vec[:, 1:2]
    z = vec[:, 2:3]
    r = jnp.sqrt(x * x + y * y + z * z)
    safe = jnp.maximum(r, 1e-6)
    inv = 1.0 / safe
    ux = x * inv
    uy = y * inv
    uz = z * inv
    s3 = 1.7320508075688772
    s5 = 2.23606797749979
    s15 = 3.872983346207417
    ones = jnp.ones_like(ux)
    sh_cols = [ones, s3 * ux, s3 * uy, s3 * uz,
               s15 * ux * uy, s15 * uy * uz,
               0.5 * s5 * (3.0 * uz * uz - 1.0),
               s15 * ux * uz, 0.5 * s15 * (ux * ux - uy * uy)]
    sh = jnp.concatenate(sh_cols, axis=1)

    n = (jax.lax.broadcasted_iota(jnp.int32, (1, _NUM_BESSEL), 1)
         .astype(jnp.float32) + 1.0)
    bes = jnp.sin(safe * (n * (math.pi / _R_MAX))) * (
        math.sqrt(2.0 / _R_MAX) * inv)
    xc = safe * (1.0 / _R_MAX)
    xc5 = xc * xc * xc * xc * xc
    env = 1.0 - 21.0 * xc5 + 35.0 * xc5 * xc - 15.0 * xc5 * xc * xc
    cut = jnp.where(xc < 1.0, env, 0.0)
    feats = bes * cut

    wh = jax.nn.relu(jnp.dot(feats, w1_ref[pl.ds(l, 1)][0],
                             preferred_element_type=jnp.float32)
                     + b1_ref[pl.ds(l, 1), :])
    w_full = (jnp.dot(wh, w2_ref[pl.ds(l, 1)][0],
                      preferred_element_type=jnp.float32)
              + b2_ref[pl.ds(l, 1), :])
    w0 = w_full[:, :_C]
    w1c = w_full[:, _C:]

    hsrc = hsrc_ref[:]
    hs0 = hsrc[:, :_C]
    scal = hs0 * sh[:, 0:1]
    for k in range(1, _K):
        scal = scal + hsrc[:, k * _C:(k + 1) * _C] * sh[:, k:k + 1]

    is_l0 = l == 0
    a = jnp.where(is_l0, w0 * hs0, w1c * scal)
    b0 = jnp.where(is_l0, jnp.zeros_like(hs0), w0 * hs0)
    parts = [a + b0]
    for k in range(1, _K):
        parts.append(a * sh[:, k:k + 1])
    msg_ref[:] = jnp.concatenate(parts, axis=1)

    def scat_body(i, carry):
        d = jnp.bitwise_and(idx_ref[base + i], 16383)
        agg_ref[pl.ds(d, 1), :] = (agg_ref[pl.ds(d, 1), :]
                                   + msg_ref[pl.ds(i, 1), :])
        return carry

    jax.lax.fori_loop(0, _EB, scat_body, 0)

    @pl.when(jnp.logical_and(l == _NUM_LAYERS - 1, b == _NB - 1))
    def _():
        apply_update()
        pooled = jnp.zeros((_G, _C), jnp.float32)
        for r in range(_NRB):
            rows = slice(r * _RB, (r + 1) * _RB)
            pooled = pooled + jax.lax.dot_general(
                boh_ref[rows, :], h_ref[rows, :_C],
                (((0,), (0,)), ((), ())),
                preferred_element_type=jnp.float32)
        t = jax.nn.relu(jnp.dot(pooled, pw1_ref[:],
                                preferred_element_type=jnp.float32)
                        + pb1_ref[:])
        out_ref[:] = jnp.dot(t, pw2_ref[:],
                             preferred_element_type=jnp.float32) + pb2_ref[:]


@jax.jit
def kernel(atoms, pos, edge_index, batch, emb, mlp_w1, mlp_b1, mlp_w2, mlp_b2,
           pred_w1, pred_b1, pred_w2, pred_b2):
    src = edge_index[0].astype(jnp.int32)
    dst = edge_index[1].astype(jnp.int32)
    vec = pos[src] - pos[dst]
    aoh = jax.nn.one_hot(atoms, emb.shape[0], dtype=jnp.float32)
    boh = jax.nn.one_hot(batch, _G, dtype=jnp.float32)
    L, C = _NUM_LAYERS, _C
    w2r = mlp_w2.reshape(L, C, C, 2).transpose(0, 1, 3, 2).reshape(L, C, 2 * C)
    b2r = mlp_b2.reshape(L, C, 2).transpose(0, 2, 1).reshape(L, 2 * C)

    packed = jnp.left_shift(src, 14) | dst
    grid = (_NUM_LAYERS, _NB)
    grid_spec = pltpu.PrefetchScalarGridSpec(
        num_scalar_prefetch=1,
        grid=grid,
        in_specs=[
            pl.BlockSpec((_EB, 3), lambda l, b, p: (b, 0)),
            pl.BlockSpec((_N, _G), lambda l, b, p: (0, 0)),
            pl.BlockSpec((_N, _G), lambda l, b, p: (0, 0)),
            pl.BlockSpec((_G, C), lambda l, b, p: (0, 0)),
            pl.BlockSpec((L, _NUM_BESSEL, C), lambda l, b, p: (0, 0, 0)),
            pl.BlockSpec((L, C), lambda l, b, p: (0, 0)),
            pl.BlockSpec((L, C, 2 * C), lambda l, b, p: (0, 0, 0)),
            pl.BlockSpec((L, 2 * C), lambda l, b, p: (0, 0)),
            pl.BlockSpec((C, C), lambda l, b, p: (0, 0)),
            pl.BlockSpec((1, C), lambda l, b, p: (0, 0)),
            pl.BlockSpec((C, 1), lambda l, b, p: (0, 0)),
            pl.BlockSpec((1, 1), lambda l, b, p: (0, 0)),
        ],
        out_specs=pl.BlockSpec((_G, 1), lambda l, b, p: (0, 0)),
        scratch_shapes=[
            pltpu.VMEM((_N, _CK), jnp.float32),
            pltpu.VMEM((_N, _CK), jnp.float32),
            pltpu.VMEM((_EB, _CK), jnp.float32),
            pltpu.VMEM((_EB, _CK), jnp.float32),
        ],
    )
    out = pl.pallas_call(
        _tfn_kernel,
        grid_spec=grid_spec,
        out_shape=jax.ShapeDtypeStruct((_G, 1), jnp.float32),
        compiler_params=pltpu.CompilerParams(
            dimension_semantics=("arbitrary", "arbitrary")),
    )(packed, vec, aoh, boh, emb,
      mlp_w1, mlp_b1, w2r, b2r,
      pred_w1, pred_b1.reshape(1, C), pred_w2, pred_b2.reshape(1, 1))
    return out
